# baseline (device time: 91216 ns/iter reference)
import jax
import jax.numpy as jnp
from jax import lax
from jax.experimental import pallas as pl
from jax.experimental.pallas import tpu as pltpu

N_DEV = 4
B, SQ, SKV = 2, 512, 512
HQ, DH = 32, 64
HL = HQ // N_DEV
DL = HL * DH
DM = 768


def kernel(x, Wq, K_ext, V_ext, Wo):
    Wq3 = Wq.reshape(DM, HQ, DH)

    def body(x_ref, wq_ref, k_ref, v_ref, wo_ref, out_ref,
             comm_ref, send_sems, recv_sems):
        my = lax.axis_index("i")
        left = lax.rem(my + N_DEV - 1, N_DEV)
        right = lax.rem(my + 1, N_DEV)

        barrier_sem = pltpu.get_barrier_semaphore()
        for nbr in (left, right):
            pl.semaphore_signal(barrier_sem, inc=1, device_id=(nbr,),
                                device_id_type=pl.DeviceIdType.MESH)
        pl.semaphore_wait(barrier_sem, 2)

        qi = lax.broadcasted_iota(jnp.int32, (SQ, SKV), 0)
        ki = lax.broadcasted_iota(jnp.int32, (SQ, SKV), 1)
        mask = (jnp.abs(qi - ki) <= 128) | (ki < 32) | (qi < 32)

        wq_loc = wq_ref[:, pl.ds(my * HL, HL), :].astype(jnp.bfloat16)
        wo_my = wo_ref[pl.ds(my * DL, DL), :].astype(jnp.bfloat16)

        acc = []
        for b in range(B):
            xb = x_ref[b].astype(jnp.bfloat16)
            ctx_cols = []
            for h in range(HL):
                qh = lax.dot(xb, wq_loc[:, h, :],
                             preferred_element_type=jnp.float32)
                kh = k_ref[b, :, h, :].astype(jnp.bfloat16)
                vh = v_ref[b, :, h, :].astype(jnp.bfloat16)
                s = lax.dot_general(qh.astype(jnp.bfloat16), kh,
                                    (((1,), (1,)), ((), ())),
                                    preferred_element_type=jnp.float32)
                s = jnp.where(mask, s * 0.125, jnp.float32(-1e9))
                s = s - jnp.max(s, axis=1, keepdims=True)
                w = jnp.exp(s)
                w = w / jnp.sum(w, axis=1, keepdims=True)
                ctx_h = lax.dot(w.astype(jnp.bfloat16), vh,
                                preferred_element_type=jnp.float32)
                ctx_cols.append(ctx_h.astype(jnp.bfloat16))
            ctx_b = jnp.concatenate(ctx_cols, axis=1)
            comm_ref[0, b] = ctx_b
            acc.append(lax.dot(ctx_b, wo_my,
                               preferred_element_type=jnp.float32))

        for h in range(N_DEV - 1):
            rdma = pltpu.make_async_remote_copy(
                src_ref=comm_ref.at[h],
                dst_ref=comm_ref.at[h + 1],
                send_sem=send_sems.at[h],
                recv_sem=recv_sems.at[h],
                device_id=(right,),
                device_id_type=pl.DeviceIdType.MESH,
            )
            rdma.start()
            rdma.wait()

            origin = lax.rem(my - h - 1 + N_DEV, N_DEV)
            wo_o = wo_ref[pl.ds(origin * DL, DL), :].astype(jnp.bfloat16)
            for b in range(B):
                chunk = comm_ref[h + 1, b]
                acc[b] = acc[b] + lax.dot(chunk, wo_o,
                                          preferred_element_type=jnp.float32)

        for b in range(B):
            out_ref[b] = acc[b]

    return pl.pallas_call(
        body,
        out_shape=jax.ShapeDtypeStruct((B, SQ, DM), jnp.float32),
        in_specs=[pl.BlockSpec(memory_space=pltpu.VMEM)] * 5,
        out_specs=pl.BlockSpec(memory_space=pltpu.VMEM),
        scratch_shapes=[
            pltpu.VMEM((N_DEV, B, SQ, DL), jnp.bfloat16),
            pltpu.SemaphoreType.DMA((N_DEV - 1,)),
            pltpu.SemaphoreType.DMA((N_DEV - 1,)),
        ],
        compiler_params=pltpu.CompilerParams(collective_id=0),
    )(x, Wq3, K_ext, V_ext, Wo)


# device time: 66396 ns/iter; 1.3738x vs baseline; 1.3738x over previous
import jax
import jax.numpy as jnp
from jax import lax
from jax.experimental import pallas as pl
from jax.experimental.pallas import tpu as pltpu

N_DEV = 4
B, SQ, SKV = 2, 512, 512
HQ, DH = 32, 64
HL = HQ // N_DEV
DL = HL * DH
DM = 768


def kernel(x, Wq, K_ext, V_ext, Wo):
    Wq3 = Wq.reshape(DM, HQ, DH)

    def body(x_ref, wq_ref, k_ref, v_ref, wo_ref, out_ref,
             my_ctx_ref, comm_ref, send_sems, recv_sems):
        my = lax.axis_index("i")
        peers = [lax.rem(my + d, N_DEV) for d in (1, 2, 3)]

        barrier_sem = pltpu.get_barrier_semaphore()
        for nbr in peers:
            pl.semaphore_signal(barrier_sem, inc=1, device_id=(nbr,),
                                device_id_type=pl.DeviceIdType.MESH)
        pl.semaphore_wait(barrier_sem, 3)

        qi = lax.broadcasted_iota(jnp.int32, (SQ, SKV), 0)
        ki = lax.broadcasted_iota(jnp.int32, (SQ, SKV), 1)
        mask = (jnp.abs(qi - ki) <= 128) | (ki < 32) | (qi < 32)
        bias = jnp.where(mask, jnp.float32(0.0), jnp.float32(-1e9))

        wq_loc = wq_ref[:, pl.ds(my * HL, HL), :].astype(jnp.bfloat16)

        sends = []
        ctx = []
        for b in range(B):
            xb = x_ref[b].astype(jnp.bfloat16)
            ctx_cols = []
            for h in range(HL):
                qh = lax.dot(xb, wq_loc[:, h, :],
                             preferred_element_type=jnp.float32)
                kh = k_ref[b, :, h, :].astype(jnp.bfloat16)
                vh = v_ref[b, :, h, :].astype(jnp.bfloat16)
                s = lax.dot_general(qh.astype(jnp.bfloat16), kh,
                                    (((1,), (1,)), ((), ())),
                                    preferred_element_type=jnp.float32)
                w = jnp.exp(s * 0.125 + bias)
                denom = jnp.sum(w, axis=1, keepdims=True)
                ctx_h = lax.dot(w.astype(jnp.bfloat16), vh,
                                preferred_element_type=jnp.float32)
                ctx_cols.append((ctx_h / denom).astype(jnp.bfloat16))
            ctx_b = jnp.concatenate(ctx_cols, axis=1)
            ctx.append(ctx_b)
            my_ctx_ref[b] = ctx_b
            for t, tgt in enumerate(peers):
                rdma = pltpu.make_async_remote_copy(
                    src_ref=my_ctx_ref.at[b],
                    dst_ref=comm_ref.at[my, b],
                    send_sem=send_sems.at[t, b],
                    recv_sem=recv_sems.at[my, b],
                    device_id=(tgt,),
                    device_id_type=pl.DeviceIdType.MESH,
                )
                rdma.start()
                sends.append(rdma)

        wo_my = wo_ref[pl.ds(my * DL, DL), :].astype(jnp.bfloat16)
        acc = [lax.dot(ctx[b], wo_my, preferred_element_type=jnp.float32)
               for b in range(B)]

        for o in peers:
            wo_o = wo_ref[pl.ds(o * DL, DL), :].astype(jnp.bfloat16)
            for b in range(B):
                recv = pltpu.make_async_remote_copy(
                    src_ref=my_ctx_ref.at[b],
                    dst_ref=comm_ref.at[o, b],
                    send_sem=send_sems.at[0, b],
                    recv_sem=recv_sems.at[o, b],
                    device_id=(o,),
                    device_id_type=pl.DeviceIdType.MESH,
                )
                recv.wait_recv()
                acc[b] = acc[b] + lax.dot(comm_ref[o, b], wo_o,
                                          preferred_element_type=jnp.float32)

        for b in range(B):
            out_ref[b] = acc[b]

        for rdma in sends:
            rdma.wait_send()

    return pl.pallas_call(
        body,
        out_shape=jax.ShapeDtypeStruct((B, SQ, DM), jnp.float32),
        in_specs=[pl.BlockSpec(memory_space=pltpu.VMEM)] * 5,
        out_specs=pl.BlockSpec(memory_space=pltpu.VMEM),
        scratch_shapes=[
            pltpu.VMEM((B, SQ, DL), jnp.bfloat16),
            pltpu.VMEM((N_DEV, B, SQ, DL), jnp.bfloat16),
            pltpu.SemaphoreType.DMA((3, B)),
            pltpu.SemaphoreType.DMA((N_DEV, B)),
        ],
        compiler_params=pltpu.CompilerParams(collective_id=0),
    )(x, Wq3, K_ext, V_ext, Wo)
